# baseline (speedup 1.0000x reference)
import functools

import jax
import jax.numpy as jnp
from jax import lax
from jax.experimental import pallas as pl
from jax.experimental.pallas import tpu as pltpu
from jax.experimental.pallas import tpu_sc as plsc

CLS = 1000
CBINS = 1024
FEAT = 64
N = 1_000_000

NC, NS, L = 2, 16, 16
NW = NC * NS

Q = 31_264
QL = N - (NW - 1) * Q

BLK = 320
NBLK = N // BLK
GPB = BLK // L
NSLOT = 2 * ((NBLK + 2 * NW - 1) // (2 * NW))
PAIRS = NSLOT // 2

_mesh = plsc.VectorSubcoreMesh(core_axis_name="c", subcore_axis_name="s")
_params = pltpu.CompilerParams(needs_layout_passes=False)


@functools.partial(
    pl.kernel,
    out_type=jax.ShapeDtypeStruct((NW, CBINS), jnp.float32),
    mesh=_mesh,
    scratch_types=[
        pltpu.VMEM((Q,), jnp.float32),
        pltpu.VMEM((L * CBINS,), jnp.float32),
        pltpu.VMEM((CBINS,), jnp.float32),
    ],
    compiler_params=_params,
)
def _hist_kernel(lbl_hbm, out_hbm, lbl_v, h_v, cnt_v):
    c = lax.axis_index("c")
    s = lax.axis_index("s")
    wid = s * NC + c

    zeros = jnp.zeros((L,), jnp.float32)

    def zero_body(j, _):
        h_v[pl.ds(j * L, L)] = zeros
        return 0

    lax.fori_loop(0, (L * CBINS) // L, zero_body, 0)

    @pl.when(wid < NW - 1)
    def _():
        pltpu.sync_copy(lbl_hbm.at[pl.ds(wid * Q, Q)], lbl_v)

    @pl.when(wid == NW - 1)
    def _():
        pltpu.sync_copy(lbl_hbm.at[pl.ds((NW - 1) * Q, QL)],
                        lbl_v.at[pl.ds(0, QL)])

    rows = lax.iota(jnp.int32, L) * CBINS
    ones = jnp.ones((L,), jnp.float32)
    ng = jnp.where(wid < NW - 1, Q // L, QL // L)

    def scat_body(g, _):
        lv = lbl_v[pl.ds(g * L, L)].astype(jnp.int32)
        plsc.addupdate_scatter(h_v, [rows + lv], ones)
        return 0

    lax.fori_loop(0, ng, scat_body, 0)

    def red_body(j, _):
        acc = h_v[pl.ds(j * L, L)]
        for r in range(1, L):
            acc = acc + h_v[pl.ds(r * CBINS + j * L, L)]
        cnt_v[pl.ds(j * L, L)] = acc
        return 0

    lax.fori_loop(0, CBINS // L, red_body, 0)
    pltpu.sync_copy(cnt_v, out_hbm.at[wid])


@functools.partial(
    pl.kernel,
    out_type=jax.ShapeDtypeStruct((NW, L), jnp.float32),
    mesh=_mesh,
    scratch_types=[
        pltpu.VMEM((CLS * FEAT,), jnp.float32),
        pltpu.VMEM((8 * CBINS,), jnp.float32),
        pltpu.VMEM((CBINS,), jnp.float32),
        pltpu.VMEM((BLK * FEAT,), jnp.float32),
        pltpu.VMEM((BLK * FEAT,), jnp.float32),
        pltpu.VMEM((BLK,), jnp.float32),
        pltpu.VMEM((BLK,), jnp.float32),
        pltpu.VMEM((L,), jnp.float32),
        pltpu.VMEM((L * FEAT,), jnp.int32),
        pltpu.SemaphoreType.DMA,
        pltpu.SemaphoreType.DMA,
    ],
    compiler_params=_params,
)
def _loss_kernel(data_hbm, lbl_hbm, cen_hbm, cnt_hbm, out_hbm,
                 cen_v, c8_v, cnt_v, dat0_v, dat1_v, lb0_v, lb1_v, res_v,
                 tbl_v, sem0, sem1):
    c = lax.axis_index("c")
    s = lax.axis_index("s")
    wid = s * NC + c

    pltpu.sync_copy(cen_hbm, cen_v)

    for chunk in range(NW // 8):
        pltpu.sync_copy(cnt_hbm.at[pl.ds(chunk * 8 * CBINS, 8 * CBINS)], c8_v)

        def comb_body(j, _, first=(chunk == 0)):
            acc = c8_v[pl.ds(j * L, L)]
            for w in range(1, 8):
                acc = acc + c8_v[pl.ds(w * CBINS + j * L, L)]
            if first:
                cnt_v[pl.ds(j * L, L)] = acc
            else:
                cnt_v[pl.ds(j * L, L)] = cnt_v[pl.ds(j * L, L)] + acc
            return 0

        lax.fori_loop(0, CBINS // L, comb_body, 0)

    siota = lax.iota(jnp.int32, L)
    riota = siota * FEAT
    for f in range(FEAT):
        tbl_v[pl.ds(f * L, L)] = riota + ((siota + f) & (FEAT - 1))
    half = jnp.float32(0.5)
    three_half = jnp.float32(1.5)
    nblk = jnp.int32(NBLK)

    bufs = ((dat0_v, lb0_v, sem0), (dat1_v, lb1_v, sem1))

    def dma_start(slot, buf):
        dat_v, lb_v, sem = bufs[buf]
        b = jnp.minimum(wid + slot * NW, nblk - 1)
        row0 = b * BLK
        pltpu.async_copy(data_hbm.at[pl.ds(row0 * FEAT, BLK * FEAT)],
                         dat_v, sem)
        pltpu.async_copy(lbl_hbm.at[pl.ds(row0, BLK)], lb_v, sem)

    def dma_wait(buf):
        dat_v, lb_v, sem = bufs[buf]
        pltpu.make_async_copy(data_hbm.at[pl.ds(0, BLK * FEAT)], dat_v,
                              sem).wait()
        pltpu.make_async_copy(lbl_hbm.at[pl.ds(0, BLK)], lb_v, sem).wait()

    def compute(slot, buf, lsum):
        dat_v, lb_v, _ = bufs[buf]
        valid = (wid + slot * NW) < nblk

        def grp_body(g, ls):
            lv = lb_v[pl.ds(g * L, L)].astype(jnp.int32)
            lv = jnp.minimum(jnp.maximum(lv, 0), CLS - 1)
            cw = plsc.load_gather(cnt_v, [lv])
            cdelta = (lv * FEAT) - riota
            dat_g = dat_v.at[pl.ds(g * (L * FEAT), L * FEAT)]
            acc = jnp.zeros((L,), jnp.float32)
            def feat_chunk(q, a):
                for fi in range(16):
                    dix = tbl_v[pl.ds(q * (16 * L) + fi * L, L)]
                    dv = plsc.load_gather(dat_g, [dix])
                    cv = plsc.load_gather(cen_v, [dix + cdelta])
                    t = dv - cv
                    a = a + t * t
                return a

            acc = lax.fori_loop(0, FEAT // 16, feat_chunk, acc)
            x = jnp.maximum(acc, jnp.float32(1e-30))
            i = plsc.bitcast(x, jnp.int32)
            i = jnp.int32(0x5F3759DF) - lax.shift_right_logical(i, 1)
            y = plsc.bitcast(i, jnp.float32)
            for _ in range(3):
                y = y * (three_half - half * x * y * y)
            dist = jnp.where(acc > 0.0, x * y, jnp.float32(0.0))
            return ls + jnp.where(valid, dist / cw, jnp.float32(0.0))

        return lax.fori_loop(0, GPB, grp_body, lsum)

    dma_start(jnp.int32(0), 0)

    def pair_body(p, lsum):
        s0 = 2 * p
        dma_start(s0 + 1, 1)
        dma_wait(0)
        lsum = compute(s0, 0, lsum)
        dma_start(s0 + 2, 0)
        dma_wait(1)
        lsum = compute(s0 + 1, 1, lsum)
        return lsum

    lsum = lax.fori_loop(0, PAIRS, pair_body,
                         jnp.zeros((L,), jnp.float32))
    dma_wait(0)
    res_v[...] = lsum
    pltpu.sync_copy(res_v, out_hbm.at[wid])


def kernel(data, label, center):
    counts = _hist_kernel(label)
    parts = _loss_kernel(data.reshape(-1), label, center.reshape(-1),
                         counts.reshape(-1))
    return jnp.sum(parts)


# speedup vs baseline: 1.0603x; 1.0603x over previous
import functools

import jax
import jax.numpy as jnp
from jax import lax
from jax.experimental import pallas as pl
from jax.experimental.pallas import tpu as pltpu
from jax.experimental.pallas import tpu_sc as plsc

CLS = 1000
CBINS = 1024
FEAT = 64
N = 1_000_000

NC, NS, L = 2, 16, 16
NW = NC * NS

Q = 31_264
QL = N - (NW - 1) * Q

BLK = 320
NBLK = N // BLK
GPB = BLK // L
NSLOT = 2 * ((NBLK + 2 * NW - 1) // (2 * NW))
PAIRS = NSLOT // 2

_mesh = plsc.VectorSubcoreMesh(core_axis_name="c", subcore_axis_name="s")
_params = pltpu.CompilerParams(needs_layout_passes=False)


@functools.partial(
    pl.kernel,
    out_type=jax.ShapeDtypeStruct((NW, CBINS), jnp.float32),
    mesh=_mesh,
    scratch_types=[
        pltpu.VMEM((Q,), jnp.float32),
        pltpu.VMEM((L * CBINS,), jnp.float32),
        pltpu.VMEM((CBINS,), jnp.float32),
    ],
    compiler_params=_params,
)
def _hist_kernel(lbl_hbm, out_hbm, lbl_v, h_v, cnt_v):
    c = lax.axis_index("c")
    s = lax.axis_index("s")
    wid = s * NC + c

    zeros = jnp.zeros((L,), jnp.float32)

    def zero_body(j, _):
        h_v[pl.ds(j * L, L)] = zeros
        return 0

    lax.fori_loop(0, (L * CBINS) // L, zero_body, 0)

    @pl.when(wid < NW - 1)
    def _():
        pltpu.sync_copy(lbl_hbm.at[pl.ds(wid * Q, Q)], lbl_v)

    @pl.when(wid == NW - 1)
    def _():
        pltpu.sync_copy(lbl_hbm.at[pl.ds((NW - 1) * Q, QL)],
                        lbl_v.at[pl.ds(0, QL)])

    rows = lax.iota(jnp.int32, L) * CBINS
    ones = jnp.ones((L,), jnp.float32)
    ng = jnp.where(wid < NW - 1, Q // L, QL // L)

    def scat_body(g, _):
        lv = lbl_v[pl.ds(g * L, L)].astype(jnp.int32)
        plsc.addupdate_scatter(h_v, [rows + lv], ones)
        return 0

    lax.fori_loop(0, ng, scat_body, 0)

    def red_body(j, _):
        acc = h_v[pl.ds(j * L, L)]
        for r in range(1, L):
            acc = acc + h_v[pl.ds(r * CBINS + j * L, L)]
        cnt_v[pl.ds(j * L, L)] = acc
        return 0

    lax.fori_loop(0, CBINS // L, red_body, 0)
    pltpu.sync_copy(cnt_v, out_hbm.at[wid])


@functools.partial(
    pl.kernel,
    out_type=jax.ShapeDtypeStruct((NW, L), jnp.float32),
    mesh=_mesh,
    scratch_types=[
        pltpu.VMEM((CLS * FEAT,), jnp.float32),
        pltpu.VMEM((8 * CBINS,), jnp.float32),
        pltpu.VMEM((CBINS,), jnp.float32),
        pltpu.VMEM((BLK * FEAT,), jnp.float32),
        pltpu.VMEM((BLK * FEAT,), jnp.float32),
        pltpu.VMEM((BLK,), jnp.float32),
        pltpu.VMEM((BLK,), jnp.float32),
        pltpu.VMEM((L,), jnp.float32),
        pltpu.VMEM((L * FEAT,), jnp.int32),
        pltpu.SemaphoreType.DMA,
        pltpu.SemaphoreType.DMA,
    ],
    compiler_params=_params,
)
def _loss_kernel(data_hbm, lbl_hbm, cen_hbm, cnt_hbm, out_hbm,
                 cen_v, c8_v, cnt_v, dat0_v, dat1_v, lb0_v, lb1_v, res_v,
                 tbl_v, sem0, sem1):
    c = lax.axis_index("c")
    s = lax.axis_index("s")
    wid = s * NC + c

    pltpu.sync_copy(cen_hbm, cen_v)

    for chunk in range(NW // 8):
        pltpu.sync_copy(cnt_hbm.at[pl.ds(chunk * 8 * CBINS, 8 * CBINS)], c8_v)

        def comb_body(j, _, first=(chunk == 0)):
            acc = c8_v[pl.ds(j * L, L)]
            for w in range(1, 8):
                acc = acc + c8_v[pl.ds(w * CBINS + j * L, L)]
            if first:
                cnt_v[pl.ds(j * L, L)] = acc
            else:
                cnt_v[pl.ds(j * L, L)] = cnt_v[pl.ds(j * L, L)] + acc
            return 0

        lax.fori_loop(0, CBINS // L, comb_body, 0)

    siota = lax.iota(jnp.int32, L)
    riota = siota * FEAT
    for f in range(FEAT):
        tbl_v[pl.ds(f * L, L)] = riota + ((siota + f) & (FEAT - 1))
    half = jnp.float32(0.5)
    three_half = jnp.float32(1.5)
    nblk = jnp.int32(NBLK)

    bufs = ((dat0_v, lb0_v, sem0), (dat1_v, lb1_v, sem1))

    def dma_start(slot, buf):
        dat_v, lb_v, sem = bufs[buf]
        b = jnp.minimum(wid + slot * NW, nblk - 1)
        row0 = b * BLK
        pltpu.async_copy(data_hbm.at[pl.ds(row0 * FEAT, BLK * FEAT)],
                         dat_v, sem)
        pltpu.async_copy(lbl_hbm.at[pl.ds(row0, BLK)], lb_v, sem)

    def dma_wait(buf):
        dat_v, lb_v, sem = bufs[buf]
        pltpu.make_async_copy(data_hbm.at[pl.ds(0, BLK * FEAT)], dat_v,
                              sem).wait()
        pltpu.make_async_copy(lbl_hbm.at[pl.ds(0, BLK)], lb_v, sem).wait()

    def compute(slot, buf, lsum):
        dat_v, lb_v, _ = bufs[buf]
        valid = (wid + slot * NW) < nblk

        def grp_body(g, ls):
            lva = lb_v[pl.ds((2 * g) * L, L)].astype(jnp.int32)
            lvb = lb_v[pl.ds((2 * g + 1) * L, L)].astype(jnp.int32)
            lva = jnp.minimum(jnp.maximum(lva, 0), CLS - 1)
            lvb = jnp.minimum(jnp.maximum(lvb, 0), CLS - 1)
            cwa = plsc.load_gather(cnt_v, [lva])
            cwb = plsc.load_gather(cnt_v, [lvb])
            cda = (lva * FEAT) - riota
            cdb = (lvb * FEAT) - riota
            dat_a = dat_v.at[pl.ds((2 * g) * (L * FEAT), L * FEAT)]
            dat_b = dat_v.at[pl.ds((2 * g + 1) * (L * FEAT), L * FEAT)]
            acca = jnp.zeros((L,), jnp.float32)
            accb = jnp.zeros((L,), jnp.float32)

            def feat_chunk(q, accs):
                a, b = accs
                for fi in range(16):
                    dix = tbl_v[pl.ds(q * (16 * L) + fi * L, L)]
                    dva = plsc.load_gather(dat_a, [dix])
                    cva = plsc.load_gather(cen_v, [dix + cda])
                    ta = dva - cva
                    a = a + ta * ta
                    dvb = plsc.load_gather(dat_b, [dix])
                    cvb = plsc.load_gather(cen_v, [dix + cdb])
                    tb = dvb - cvb
                    b = b + tb * tb
                return (a, b)

            acca, accb = lax.fori_loop(0, FEAT // 16, feat_chunk,
                                       (acca, accb))

            def finish(acc, cw):
                x = jnp.maximum(acc, jnp.float32(1e-30))
                i = plsc.bitcast(x, jnp.int32)
                i = jnp.int32(0x5F3759DF) - lax.shift_right_logical(i, 1)
                y = plsc.bitcast(i, jnp.float32)
                for _ in range(3):
                    y = y * (three_half - half * x * y * y)
                dist = jnp.where(acc > 0.0, x * y, jnp.float32(0.0))
                return jnp.where(valid, dist / cw, jnp.float32(0.0))

            return ls + finish(acca, cwa) + finish(accb, cwb)

        return lax.fori_loop(0, GPB // 2, grp_body, lsum)

    dma_start(jnp.int32(0), 0)

    def pair_body(p, lsum):
        s0 = 2 * p
        dma_start(s0 + 1, 1)
        dma_wait(0)
        lsum = compute(s0, 0, lsum)
        dma_start(s0 + 2, 0)
        dma_wait(1)
        lsum = compute(s0 + 1, 1, lsum)
        return lsum

    lsum = lax.fori_loop(0, PAIRS, pair_body,
                         jnp.zeros((L,), jnp.float32))
    dma_wait(0)
    res_v[...] = lsum
    pltpu.sync_copy(res_v, out_hbm.at[wid])


def kernel(data, label, center):
    counts = _hist_kernel(label)
    parts = _loss_kernel(data.reshape(-1), label, center.reshape(-1),
                         counts.reshape(-1))
    return jnp.sum(parts)
